# Initial kernel scaffold; baseline (speedup 1.0000x reference)
import jax
import jax.numpy as jnp
from jax.experimental import pallas as pl


def kernel(h, edge_index, W_msg, b_msg, W_ih, b_ih, W_hh, b_hh, W_cls, b_cls):
    raise NotImplementedError("write your pallas kernel here")


# speedup vs baseline: 3.3768x; 3.3768x over previous
import functools

import jax
import jax.numpy as jnp
from jax import lax
from jax.experimental import pallas as pl
from jax.experimental.pallas import tpu as pltpu
from jax.experimental.pallas import tpu_sc as plsc

N = 10000
E = 320000
H = 128
C = 10
STEPS = 3

NC = 2
NS = 16

CHUNK = 128
KS = 8
K = 2
E_PAD = 327680
NCHUNKS = E_PAD // CHUNK
CH_PER_CORE = NCHUNKS // NC
CH_PER_SUB = CH_PER_CORE // NS
GROUPS = CH_PER_SUB // KS
N_OUT = 10240
ROWS_PER_SUB = N_OUT // NS

BN = 1000
NB = N // BN


@functools.cache
def _sc_scatter_kernel():
    return functools.partial(
        pl.kernel,
        out_type=jax.ShapeDtypeStruct((NC, N_OUT, H), jnp.float32),
        mesh=plsc.VectorSubcoreMesh(core_axis_name="c", subcore_axis_name="s",
                                    num_cores=NC, num_subcores=NS),
        scratch_types=[
            pltpu.VMEM_SHARED((N_OUT, H), jnp.float32),
            pltpu.VMEM((KS, CHUNK), jnp.int32),
            pltpu.VMEM((KS, CHUNK), jnp.int32),
            pltpu.VMEM((K * CHUNK, H), jnp.float32),
            pltpu.SemaphoreType.DMA,
            pltpu.SemaphoreType.DMA,
        ],
    )(_sc_scatter_body)


def _sc_scatter_body(t_hbm, src_hbm, dst_hbm, out_hbm, m_sh, src_v, dst_v,
                     rows_v, gsem, ssem):
    c = lax.axis_index("c")
    s = lax.axis_index("s")
    r0 = s * ROWS_PER_SUB

    def zbody(i, _):
        for j in range(H // 16):
            rows_v[i, pl.ds(j * 16, 16)] = jnp.zeros((16,), jnp.float32)
        return 0

    lax.fori_loop(0, K * CHUNK, zbody, 0)
    pltpu.sync_copy(rows_v, m_sh.at[pl.ds(r0, K * CHUNK)])
    pltpu.sync_copy(rows_v, m_sh.at[pl.ds(r0 + K * CHUNK, K * CHUNK)])
    pltpu.sync_copy(rows_v.at[pl.ds(0, ROWS_PER_SUB - 2 * K * CHUNK)],
                    m_sh.at[pl.ds(r0 + 2 * K * CHUNK,
                                  ROWS_PER_SUB - 2 * K * CHUNK)])
    plsc.subcore_barrier()

    def gbody(g, _):
        ch0 = c * CH_PER_CORE + s * CH_PER_SUB + g * KS
        pltpu.sync_copy(src_hbm.at[pl.ds(ch0, KS)], src_v)
        pltpu.sync_copy(dst_hbm.at[pl.ds(ch0, KS)], dst_v)
        for half in range(KS // K):
            gs = [
                pltpu.async_copy(t_hbm.at[src_v.at[half * K + kk]],
                                 rows_v.at[pl.ds(kk * CHUNK, CHUNK)], gsem)
                for kk in range(K)
            ]
            for hd in gs:
                hd.wait()
            ss = [
                pltpu.async_copy(rows_v.at[pl.ds(kk * CHUNK, CHUNK)],
                                 m_sh.at[dst_v.at[half * K + kk]], ssem,
                                 add=True)
                for kk in range(K)
            ]
            for hd in ss:
                hd.wait()
        return 0

    lax.fori_loop(0, GROUPS, gbody, 0)
    plsc.subcore_barrier()
    pltpu.sync_copy(m_sh.at[pl.ds(r0, ROWS_PER_SUB)],
                    out_hbm.at[c, pl.ds(r0, ROWS_PER_SUB)])


def _sigmoid(v):
    return 1.0 / (1.0 + jnp.exp(-v))


def _msg_body(x_ref, wm_ref, bm_ref, t_ref):
    t_ref[...] = (jnp.dot(x_ref[...], wm_ref[...],
                          preferred_element_type=jnp.float32) + bm_ref[...])


_msg_tc = pl.pallas_call(
    _msg_body,
    grid=(NB,),
    in_specs=[
        pl.BlockSpec((BN, H), lambda i: (i, 0)),
        pl.BlockSpec((H, H), lambda i: (0, 0)),
        pl.BlockSpec((1, H), lambda i: (0, 0)),
    ],
    out_specs=pl.BlockSpec((BN, H), lambda i: (i, 0)),
    out_shape=jax.ShapeDtypeStruct((N, H), jnp.float32),
)


def _gru(m, x, wih_ref, bih_ref, whh_ref, bhh_ref):
    gi = jnp.dot(m, wih_ref[...], preferred_element_type=jnp.float32) + bih_ref[...]
    gh = jnp.dot(x, whh_ref[...], preferred_element_type=jnp.float32) + bhh_ref[...]
    r = _sigmoid(gi[:, :H] + gh[:, :H])
    z = _sigmoid(gi[:, H:2 * H] + gh[:, H:2 * H])
    n = jnp.tanh(gi[:, 2 * H:] + r * gh[:, 2 * H:])
    return (1.0 - z) * n + z * x


def _gru_msg_body(m2_ref, x_ref, wih_ref, bih_ref, whh_ref, bhh_ref,
                  wm_ref, bm_ref, xn_ref, t_ref):
    m = m2_ref[0] + m2_ref[1]
    xn = _gru(m, x_ref[...], wih_ref, bih_ref, whh_ref, bhh_ref)
    xn_ref[...] = xn
    t_ref[...] = (jnp.dot(xn, wm_ref[...],
                          preferred_element_type=jnp.float32) + bm_ref[...])


_gru_msg_tc = pl.pallas_call(
    _gru_msg_body,
    grid=(NB,),
    in_specs=[
        pl.BlockSpec((NC, BN, H), lambda i: (0, i, 0)),
        pl.BlockSpec((BN, H), lambda i: (i, 0)),
        pl.BlockSpec((H, 3 * H), lambda i: (0, 0)),
        pl.BlockSpec((1, 3 * H), lambda i: (0, 0)),
        pl.BlockSpec((H, 3 * H), lambda i: (0, 0)),
        pl.BlockSpec((1, 3 * H), lambda i: (0, 0)),
        pl.BlockSpec((H, H), lambda i: (0, 0)),
        pl.BlockSpec((1, H), lambda i: (0, 0)),
    ],
    out_specs=[
        pl.BlockSpec((BN, H), lambda i: (i, 0)),
        pl.BlockSpec((BN, H), lambda i: (i, 0)),
    ],
    out_shape=[
        jax.ShapeDtypeStruct((N, H), jnp.float32),
        jax.ShapeDtypeStruct((N, H), jnp.float32),
    ],
)


def _gru_read_body(m2_ref, x_ref, wih_ref, bih_ref, whh_ref, bhh_ref,
                   wc_ref, bc_ref, out_ref, maxv):
    i = pl.program_id(0)
    m = m2_ref[0] + m2_ref[1]
    xn = _gru(m, x_ref[...], wih_ref, bih_ref, whh_ref, bhh_ref)
    part = jnp.max(xn, axis=0, keepdims=True)

    @pl.when(i == 0)
    def _init():
        maxv[...] = part

    @pl.when(i > 0)
    def _acc():
        maxv[...] = jnp.maximum(maxv[...], part)

    @pl.when(i == NB - 1)
    def _fin():
        out_ref[...] = (jnp.dot(maxv[...], wc_ref[...],
                                preferred_element_type=jnp.float32) + bc_ref[...])


_gru_read_tc = pl.pallas_call(
    _gru_read_body,
    grid=(NB,),
    in_specs=[
        pl.BlockSpec((NC, BN, H), lambda i: (0, i, 0)),
        pl.BlockSpec((BN, H), lambda i: (i, 0)),
        pl.BlockSpec((H, 3 * H), lambda i: (0, 0)),
        pl.BlockSpec((1, 3 * H), lambda i: (0, 0)),
        pl.BlockSpec((H, 3 * H), lambda i: (0, 0)),
        pl.BlockSpec((1, 3 * H), lambda i: (0, 0)),
        pl.BlockSpec((H, H), lambda i: (0, 0)),
        pl.BlockSpec((1, H), lambda i: (0, 0)),
    ],
    out_specs=pl.BlockSpec((1, H), lambda i: (0, 0)),
    out_shape=jax.ShapeDtypeStruct((1, H), jnp.float32),
    scratch_shapes=[pltpu.VMEM((1, H), jnp.float32)],
)


def kernel(h, edge_index, W_msg, b_msg, W_ih, b_ih, W_hh, b_hh, W_cls, b_cls):
    src = edge_index[0]
    dst = edge_index[1]
    pad = E_PAD - E
    srcp = jnp.concatenate([src, jnp.zeros((pad,), jnp.int32)]).reshape(NCHUNKS, CHUNK)
    dstp = jnp.concatenate([dst, jnp.full((pad,), N, jnp.int32)]).reshape(NCHUNKS, CHUNK)

    wm = W_msg.T
    wih = W_ih.T
    whh = W_hh.T
    bm = b_msg.reshape(1, H)
    bih = b_ih.reshape(1, 3 * H)
    bhh = b_hh.reshape(1, 3 * H)
    wc = jnp.pad(W_cls.T, ((0, 0), (0, H - C)))
    bc = jnp.pad(b_cls.reshape(1, C), ((0, 0), (0, H - C)))

    x = h
    t = _msg_tc(x, wm, bm)
    for step in range(STEPS):
        m2 = _sc_scatter_kernel()(t, srcp, dstp)
        if step < STEPS - 1:
            x, t = _gru_msg_tc(m2, x, wih, bih, whh, bhh, wm, bm)
        else:
            logits = _gru_read_tc(m2, x, wih, bih, whh, bhh, wc, bc)
    return logits[:, :C]


# speedup vs baseline: 3.5544x; 1.0526x over previous
import functools

import jax
import jax.numpy as jnp
from jax import lax
from jax.experimental import pallas as pl
from jax.experimental.pallas import tpu as pltpu
from jax.experimental.pallas import tpu_sc as plsc

N = 10000
E = 320000
H = 128
C = 10
STEPS = 3

NC = 2
NS = 16

CHUNK = 128
KS = 8
K = 2
E_PAD = 327680
NCHUNKS = E_PAD // CHUNK
CH_PER_CORE = NCHUNKS // NC
CH_PER_SUB = CH_PER_CORE // NS
GROUPS = CH_PER_SUB // KS
N_OUT = 10240
ROWS_PER_SUB = N_OUT // NS

BN = 1000
NB = N // BN


@functools.cache
def _sc_scatter_kernel():
    return functools.partial(
        pl.kernel,
        out_type=jax.ShapeDtypeStruct((NC, N_OUT, H), jnp.float32),
        mesh=plsc.VectorSubcoreMesh(core_axis_name="c", subcore_axis_name="s",
                                    num_cores=NC, num_subcores=NS),
        scratch_types=[
            pltpu.VMEM_SHARED((N_OUT, H), jnp.float32),
            pltpu.VMEM((KS, CHUNK), jnp.int32),
            pltpu.VMEM((KS, CHUNK), jnp.int32),
            pltpu.VMEM((CHUNK, H), jnp.float32),
            pltpu.VMEM((CHUNK, H), jnp.float32),
            pltpu.SemaphoreType.DMA,
            pltpu.SemaphoreType.DMA,
        ],
    )(_sc_scatter_body)


def _sc_scatter_body(t_hbm, src_hbm, dst_hbm, out_hbm, m_sh, src_v, dst_v,
                     rows_a, rows_b, gsem, ssem):
    c = lax.axis_index("c")
    s = lax.axis_index("s")
    r0 = s * ROWS_PER_SUB
    bufs = (rows_a, rows_b)

    def zbody(i, _):
        for j in range(H // 16):
            rows_a[i, pl.ds(j * 16, 16)] = jnp.zeros((16,), jnp.float32)
        return 0

    lax.fori_loop(0, CHUNK, zbody, 0)
    for q in range(ROWS_PER_SUB // CHUNK):
        pltpu.sync_copy(rows_a, m_sh.at[pl.ds(r0 + q * CHUNK, CHUNK)])
    plsc.subcore_barrier()

    def gbody(g, _):
        ch0 = c * CH_PER_CORE + s * CH_PER_SUB + g * KS
        pltpu.sync_copy(src_hbm.at[pl.ds(ch0, KS)], src_v)
        pltpu.sync_copy(dst_hbm.at[pl.ds(ch0, KS)], dst_v)
        scat = [None, None]
        for kk in range(KS):
            b = kk % 2
            if scat[b] is not None:
                scat[b].wait()
            gh = pltpu.async_copy(t_hbm.at[src_v.at[kk]], bufs[b], gsem)
            gh.wait()
            scat[b] = pltpu.async_copy(bufs[b], m_sh.at[dst_v.at[kk]], ssem,
                                       add=True)
        for hd in scat:
            hd.wait()
        return 0

    lax.fori_loop(0, GROUPS, gbody, 0)
    plsc.subcore_barrier()
    pltpu.sync_copy(m_sh.at[pl.ds(r0, ROWS_PER_SUB)],
                    out_hbm.at[c, pl.ds(r0, ROWS_PER_SUB)])


def _sigmoid(v):
    return 1.0 / (1.0 + jnp.exp(-v))


def _msg_body(x_ref, wm_ref, bm_ref, t_ref):
    t_ref[...] = (jnp.dot(x_ref[...], wm_ref[...],
                          preferred_element_type=jnp.float32) + bm_ref[...])


_msg_tc = pl.pallas_call(
    _msg_body,
    grid=(NB,),
    in_specs=[
        pl.BlockSpec((BN, H), lambda i: (i, 0)),
        pl.BlockSpec((H, H), lambda i: (0, 0)),
        pl.BlockSpec((1, H), lambda i: (0, 0)),
    ],
    out_specs=pl.BlockSpec((BN, H), lambda i: (i, 0)),
    out_shape=jax.ShapeDtypeStruct((N, H), jnp.float32),
)


def _gru(m, x, wih_ref, bih_ref, whh_ref, bhh_ref):
    gi = jnp.dot(m, wih_ref[...], preferred_element_type=jnp.float32) + bih_ref[...]
    gh = jnp.dot(x, whh_ref[...], preferred_element_type=jnp.float32) + bhh_ref[...]
    r = _sigmoid(gi[:, :H] + gh[:, :H])
    z = _sigmoid(gi[:, H:2 * H] + gh[:, H:2 * H])
    n = jnp.tanh(gi[:, 2 * H:] + r * gh[:, 2 * H:])
    return (1.0 - z) * n + z * x


def _gru_msg_body(m2_ref, x_ref, wih_ref, bih_ref, whh_ref, bhh_ref,
                  wm_ref, bm_ref, xn_ref, t_ref):
    m = m2_ref[0] + m2_ref[1]
    xn = _gru(m, x_ref[...], wih_ref, bih_ref, whh_ref, bhh_ref)
    xn_ref[...] = xn
    t_ref[...] = (jnp.dot(xn, wm_ref[...],
                          preferred_element_type=jnp.float32) + bm_ref[...])


_gru_msg_tc = pl.pallas_call(
    _gru_msg_body,
    grid=(NB,),
    in_specs=[
        pl.BlockSpec((NC, BN, H), lambda i: (0, i, 0)),
        pl.BlockSpec((BN, H), lambda i: (i, 0)),
        pl.BlockSpec((H, 3 * H), lambda i: (0, 0)),
        pl.BlockSpec((1, 3 * H), lambda i: (0, 0)),
        pl.BlockSpec((H, 3 * H), lambda i: (0, 0)),
        pl.BlockSpec((1, 3 * H), lambda i: (0, 0)),
        pl.BlockSpec((H, H), lambda i: (0, 0)),
        pl.BlockSpec((1, H), lambda i: (0, 0)),
    ],
    out_specs=[
        pl.BlockSpec((BN, H), lambda i: (i, 0)),
        pl.BlockSpec((BN, H), lambda i: (i, 0)),
    ],
    out_shape=[
        jax.ShapeDtypeStruct((N, H), jnp.float32),
        jax.ShapeDtypeStruct((N, H), jnp.float32),
    ],
)


def _gru_read_body(m2_ref, x_ref, wih_ref, bih_ref, whh_ref, bhh_ref,
                   wc_ref, bc_ref, out_ref, maxv):
    i = pl.program_id(0)
    m = m2_ref[0] + m2_ref[1]
    xn = _gru(m, x_ref[...], wih_ref, bih_ref, whh_ref, bhh_ref)
    part = jnp.max(xn, axis=0, keepdims=True)

    @pl.when(i == 0)
    def _init():
        maxv[...] = part

    @pl.when(i > 0)
    def _acc():
        maxv[...] = jnp.maximum(maxv[...], part)

    @pl.when(i == NB - 1)
    def _fin():
        out_ref[...] = (jnp.dot(maxv[...], wc_ref[...],
                                preferred_element_type=jnp.float32) + bc_ref[...])


_gru_read_tc = pl.pallas_call(
    _gru_read_body,
    grid=(NB,),
    in_specs=[
        pl.BlockSpec((NC, BN, H), lambda i: (0, i, 0)),
        pl.BlockSpec((BN, H), lambda i: (i, 0)),
        pl.BlockSpec((H, 3 * H), lambda i: (0, 0)),
        pl.BlockSpec((1, 3 * H), lambda i: (0, 0)),
        pl.BlockSpec((H, 3 * H), lambda i: (0, 0)),
        pl.BlockSpec((1, 3 * H), lambda i: (0, 0)),
        pl.BlockSpec((H, H), lambda i: (0, 0)),
        pl.BlockSpec((1, H), lambda i: (0, 0)),
    ],
    out_specs=pl.BlockSpec((1, H), lambda i: (0, 0)),
    out_shape=jax.ShapeDtypeStruct((1, H), jnp.float32),
    scratch_shapes=[pltpu.VMEM((1, H), jnp.float32)],
)


def kernel(h, edge_index, W_msg, b_msg, W_ih, b_ih, W_hh, b_hh, W_cls, b_cls):
    src = edge_index[0]
    dst = edge_index[1]
    pad = E_PAD - E
    srcp = jnp.concatenate([src, jnp.zeros((pad,), jnp.int32)]).reshape(NCHUNKS, CHUNK)
    dstp = jnp.concatenate([dst, jnp.full((pad,), N, jnp.int32)]).reshape(NCHUNKS, CHUNK)

    wm = W_msg.T
    wih = W_ih.T
    whh = W_hh.T
    bm = b_msg.reshape(1, H)
    bih = b_ih.reshape(1, 3 * H)
    bhh = b_hh.reshape(1, 3 * H)
    wc = jnp.pad(W_cls.T, ((0, 0), (0, H - C)))
    bc = jnp.pad(b_cls.reshape(1, C), ((0, 0), (0, H - C)))

    x = h
    t = _msg_tc(x, wm, bm)
    for step in range(STEPS):
        m2 = _sc_scatter_kernel()(t, srcp, dstp)
        if step < STEPS - 1:
            x, t = _gru_msg_tc(m2, x, wih, bih, whh, bhh, wm, bm)
        else:
            logits = _gru_read_tc(m2, x, wih, bih, whh, bhh, wc, bc)
    return logits[:, :C]


# speedup vs baseline: 3.5566x; 1.0006x over previous
import functools

import jax
import jax.numpy as jnp
from jax import lax
from jax.experimental import pallas as pl
from jax.experimental.pallas import tpu as pltpu
from jax.experimental.pallas import tpu_sc as plsc

N = 10000
E = 320000
H = 128
C = 10
STEPS = 3

NC = 2
NS = 16

CHUNK = 128
KS = 8
K = 2
E_PAD = 327680
NCHUNKS = E_PAD // CHUNK
CH_PER_CORE = NCHUNKS // NC
CH_PER_SUB = CH_PER_CORE // NS
GROUPS = CH_PER_SUB // KS
N_OUT = 10240
ROWS_PER_SUB = N_OUT // NS

BN = 1000
NB = N // BN


@functools.cache
def _sc_scatter_kernel():
    return functools.partial(
        pl.kernel,
        out_type=jax.ShapeDtypeStruct((NC, N_OUT, H), jnp.float32),
        mesh=plsc.VectorSubcoreMesh(core_axis_name="c", subcore_axis_name="s",
                                    num_cores=NC, num_subcores=NS),
        scratch_types=[
            pltpu.VMEM_SHARED((N_OUT, H), jnp.float32),
            pltpu.VMEM((KS, CHUNK), jnp.int32),
            pltpu.VMEM((KS, CHUNK), jnp.int32),
            pltpu.VMEM((CHUNK, H), jnp.float32),
            pltpu.VMEM((CHUNK, H), jnp.float32),
            pltpu.SemaphoreType.DMA,
            pltpu.SemaphoreType.DMA,
        ],
    )(_sc_scatter_body)


def _sc_scatter_body(t_hbm, src_hbm, dst_hbm, out_hbm, m_sh, src_v, dst_v,
                     rows_a, rows_b, gsem, ssem):
    c = lax.axis_index("c")
    s = lax.axis_index("s")
    r0 = s * ROWS_PER_SUB
    bufs = (rows_a, rows_b)

    def zbody(i, _):
        for j in range(H // 16):
            rows_a[i, pl.ds(j * 16, 16)] = jnp.zeros((16,), jnp.float32)
        return 0

    lax.fori_loop(0, CHUNK, zbody, 0)
    for q in range(ROWS_PER_SUB // CHUNK):
        pltpu.sync_copy(rows_a, m_sh.at[pl.ds(r0 + q * CHUNK, CHUNK)])
    plsc.subcore_barrier()

    def gbody(g, _):
        ch0 = c * CH_PER_CORE + s * CH_PER_SUB + g * KS
        pltpu.sync_copy(src_hbm.at[pl.ds(ch0, KS)], src_v)
        pltpu.sync_copy(dst_hbm.at[pl.ds(ch0, KS)], dst_v)
        scat = [None, None]
        for kk in range(KS):
            b = kk % 2
            if scat[b] is not None:
                scat[b].wait()
            gh = pltpu.async_copy(t_hbm.at[src_v.at[kk]], bufs[b], gsem)
            gh.wait()
            scat[b] = pltpu.async_copy(bufs[b], m_sh.at[dst_v.at[kk]], ssem,
                                       add=True)
        for hd in scat:
            hd.wait()
        return 0

    lax.fori_loop(0, GROUPS, gbody, 0)
    plsc.subcore_barrier()
    pltpu.sync_copy(m_sh.at[pl.ds(r0, ROWS_PER_SUB)],
                    out_hbm.at[c, pl.ds(r0, ROWS_PER_SUB)])


def _sigmoid(v):
    return 1.0 / (1.0 + jnp.exp(-v))


def _msg_body(x_ref, wm_ref, bm_ref, t_ref):
    t_ref[...] = (jnp.dot(x_ref[...], wm_ref[...],
                          preferred_element_type=jnp.float32) + bm_ref[...])


_msg_tc = pl.pallas_call(
    _msg_body,
    grid=(NB,),
    in_specs=[
        pl.BlockSpec((BN, H), lambda i: (i, 0)),
        pl.BlockSpec((H, H), lambda i: (0, 0)),
        pl.BlockSpec((1, H), lambda i: (0, 0)),
    ],
    out_specs=pl.BlockSpec((BN, H), lambda i: (i, 0)),
    out_shape=jax.ShapeDtypeStruct((N, H), jnp.float32),
)


def _gru(m, x, wih_ref, bih_ref, whh_ref, bhh_ref):
    gi = jnp.dot(m, wih_ref[...], preferred_element_type=jnp.float32) + bih_ref[...]
    gh = jnp.dot(x, whh_ref[...], preferred_element_type=jnp.float32) + bhh_ref[...]
    r = _sigmoid(gi[:, :H] + gh[:, :H])
    z = _sigmoid(gi[:, H:2 * H] + gh[:, H:2 * H])
    n = jnp.tanh(gi[:, 2 * H:] + r * gh[:, 2 * H:])
    return (1.0 - z) * n + z * x


def _gru_msg_body(m2_ref, x_ref, wih_ref, bih_ref, whh_ref, bhh_ref,
                  wm_ref, bm_ref, xn_ref, t_ref):
    m = m2_ref[0] + m2_ref[1]
    xn = _gru(m, x_ref[...], wih_ref, bih_ref, whh_ref, bhh_ref)
    xn_ref[...] = xn
    t_ref[...] = (jnp.dot(xn, wm_ref[...],
                          preferred_element_type=jnp.float32) + bm_ref[...])


_gru_msg_tc = pl.pallas_call(
    _gru_msg_body,
    grid=(NB,),
    in_specs=[
        pl.BlockSpec((NC, BN, H), lambda i: (0, i, 0)),
        pl.BlockSpec((BN, H), lambda i: (i, 0)),
        pl.BlockSpec((H, 3 * H), lambda i: (0, 0)),
        pl.BlockSpec((1, 3 * H), lambda i: (0, 0)),
        pl.BlockSpec((H, 3 * H), lambda i: (0, 0)),
        pl.BlockSpec((1, 3 * H), lambda i: (0, 0)),
        pl.BlockSpec((H, H), lambda i: (0, 0)),
        pl.BlockSpec((1, H), lambda i: (0, 0)),
    ],
    out_specs=[
        pl.BlockSpec((BN, H), lambda i: (i, 0)),
        pl.BlockSpec((BN, H), lambda i: (i, 0)),
    ],
    out_shape=[
        jax.ShapeDtypeStruct((N, H), jnp.float32),
        jax.ShapeDtypeStruct((N, H), jnp.float32),
    ],
)


def _gru_read_body(m2_ref, x_ref, wih_ref, bih_ref, whh_ref, bhh_ref,
                   wc_ref, bc_ref, out_ref, maxv):
    i = pl.program_id(0)
    m = m2_ref[0] + m2_ref[1]
    xn = _gru(m, x_ref[...], wih_ref, bih_ref, whh_ref, bhh_ref)
    part = jnp.max(xn, axis=0, keepdims=True)

    @pl.when(i == 0)
    def _init():
        maxv[...] = part

    @pl.when(i > 0)
    def _acc():
        maxv[...] = jnp.maximum(maxv[...], part)

    @pl.when(i == NB - 1)
    def _fin():
        out_ref[...] = (jnp.dot(maxv[...], wc_ref[...],
                                preferred_element_type=jnp.float32) + bc_ref[...])


_gru_read_tc = pl.pallas_call(
    _gru_read_body,
    grid=(NB,),
    in_specs=[
        pl.BlockSpec((NC, BN, H), lambda i: (0, i, 0)),
        pl.BlockSpec((BN, H), lambda i: (i, 0)),
        pl.BlockSpec((H, 3 * H), lambda i: (0, 0)),
        pl.BlockSpec((1, 3 * H), lambda i: (0, 0)),
        pl.BlockSpec((H, 3 * H), lambda i: (0, 0)),
        pl.BlockSpec((1, 3 * H), lambda i: (0, 0)),
        pl.BlockSpec((H, H), lambda i: (0, 0)),
        pl.BlockSpec((1, H), lambda i: (0, 0)),
    ],
    out_specs=pl.BlockSpec((1, H), lambda i: (0, 0)),
    out_shape=jax.ShapeDtypeStruct((1, H), jnp.float32),
    scratch_shapes=[pltpu.VMEM((1, H), jnp.float32)],
)


def kernel(h, edge_index, W_msg, b_msg, W_ih, b_ih, W_hh, b_hh, W_cls, b_cls):
    src = edge_index[0]
    dst = edge_index[1]
    pad = E_PAD - E
    srcp = jnp.concatenate([src, jnp.zeros((pad,), jnp.int32)]).reshape(NCHUNKS, CHUNK)
    dst_pad = N + (jnp.arange(pad, dtype=jnp.int32) % (N_OUT - N))
    dstp = jnp.concatenate([dst, dst_pad]).reshape(NCHUNKS, CHUNK)

    wm = W_msg.T
    wih = W_ih.T
    whh = W_hh.T
    bm = b_msg.reshape(1, H)
    bih = b_ih.reshape(1, 3 * H)
    bhh = b_hh.reshape(1, 3 * H)
    wc = jnp.pad(W_cls.T, ((0, 0), (0, H - C)))
    bc = jnp.pad(b_cls.reshape(1, C), ((0, 0), (0, H - C)))

    x = h
    t = _msg_tc(x, wm, bm)
    for step in range(STEPS):
        m2 = _sc_scatter_kernel()(t, srcp, dstp)
        if step < STEPS - 1:
            x, t = _gru_msg_tc(m2, x, wih, bih, whh, bhh, wm, bm)
        else:
            logits = _gru_read_tc(m2, x, wih, bih, whh, bhh, wc, bc)
    return logits[:, :C]


# speedup vs baseline: 3.7830x; 1.0636x over previous
import functools

import jax
import jax.numpy as jnp
from jax import lax
from jax.experimental import pallas as pl
from jax.experimental.pallas import tpu as pltpu
from jax.experimental.pallas import tpu_sc as plsc

N = 10000
E = 320000
H = 128
C = 10
STEPS = 3

NC = 2
NS = 16

CHUNK = 128
KS = 16
K = 2
E_PAD = 327680
NCHUNKS = E_PAD // CHUNK
CH_PER_CORE = NCHUNKS // NC
CH_PER_SUB = CH_PER_CORE // NS
GROUPS = CH_PER_SUB // KS
N_OUT = 10240
ROWS_PER_SUB = N_OUT // NS

BN = 1000
NB = N // BN


@functools.cache
def _sc_scatter_kernel():
    return functools.partial(
        pl.kernel,
        out_type=jax.ShapeDtypeStruct((NC, N_OUT, H), jnp.float32),
        mesh=plsc.VectorSubcoreMesh(core_axis_name="c", subcore_axis_name="s",
                                    num_cores=NC, num_subcores=NS),
        scratch_types=[
            pltpu.VMEM_SHARED((N_OUT, H), jnp.float32),
            pltpu.VMEM((KS, CHUNK), jnp.int32),
            pltpu.VMEM((KS, CHUNK), jnp.int32),
            pltpu.VMEM((CHUNK, H), jnp.float32),
            pltpu.VMEM((CHUNK, H), jnp.float32),
            pltpu.SemaphoreType.DMA,
            pltpu.SemaphoreType.DMA,
        ],
    )(_sc_scatter_body)


def _sc_scatter_body(t_hbm, src_hbm, dst_hbm, out_hbm, m_sh, src_v, dst_v,
                     rows_a, rows_b, gsem, ssem):
    c = lax.axis_index("c")
    s = lax.axis_index("s")
    r0 = s * ROWS_PER_SUB
    bufs = (rows_a, rows_b)

    def zbody(i, _):
        for j in range(H // 16):
            rows_a[i, pl.ds(j * 16, 16)] = jnp.zeros((16,), jnp.float32)
        return 0

    lax.fori_loop(0, CHUNK, zbody, 0)
    for q in range(ROWS_PER_SUB // CHUNK):
        pltpu.sync_copy(rows_a, m_sh.at[pl.ds(r0 + q * CHUNK, CHUNK)])
    plsc.subcore_barrier()

    def gbody(g, _):
        ch0 = c * CH_PER_CORE + s * CH_PER_SUB + g * KS
        pltpu.sync_copy(src_hbm.at[pl.ds(ch0, KS)], src_v)
        pltpu.sync_copy(dst_hbm.at[pl.ds(ch0, KS)], dst_v)
        gath = [pltpu.async_copy(t_hbm.at[src_v.at[0]], bufs[0], gsem), None]
        scat = [None, None]
        for kk in range(KS):
            b = kk % 2
            nb = 1 - b
            if kk + 1 < KS:
                if scat[nb] is not None:
                    scat[nb].wait()
                gath[nb] = pltpu.async_copy(t_hbm.at[src_v.at[kk + 1]],
                                            bufs[nb], gsem)
            gath[b].wait()
            scat[b] = pltpu.async_copy(bufs[b], m_sh.at[dst_v.at[kk]], ssem,
                                       add=True)
        scat[0].wait()
        scat[1].wait()
        return 0

    lax.fori_loop(0, GROUPS, gbody, 0)
    plsc.subcore_barrier()
    pltpu.sync_copy(m_sh.at[pl.ds(r0, ROWS_PER_SUB)],
                    out_hbm.at[c, pl.ds(r0, ROWS_PER_SUB)])


def _sigmoid(v):
    return 1.0 / (1.0 + jnp.exp(-v))


def _msg_body(x_ref, wm_ref, bm_ref, t_ref):
    t_ref[...] = (jnp.dot(x_ref[...], wm_ref[...],
                          preferred_element_type=jnp.float32) + bm_ref[...])


_msg_tc = pl.pallas_call(
    _msg_body,
    grid=(NB,),
    in_specs=[
        pl.BlockSpec((BN, H), lambda i: (i, 0)),
        pl.BlockSpec((H, H), lambda i: (0, 0)),
        pl.BlockSpec((1, H), lambda i: (0, 0)),
    ],
    out_specs=pl.BlockSpec((BN, H), lambda i: (i, 0)),
    out_shape=jax.ShapeDtypeStruct((N, H), jnp.float32),
)


def _gru(m, x, wih_ref, bih_ref, whh_ref, bhh_ref):
    gi = jnp.dot(m, wih_ref[...], preferred_element_type=jnp.float32) + bih_ref[...]
    gh = jnp.dot(x, whh_ref[...], preferred_element_type=jnp.float32) + bhh_ref[...]
    r = _sigmoid(gi[:, :H] + gh[:, :H])
    z = _sigmoid(gi[:, H:2 * H] + gh[:, H:2 * H])
    n = jnp.tanh(gi[:, 2 * H:] + r * gh[:, 2 * H:])
    return (1.0 - z) * n + z * x


def _gru_msg_body(m2_ref, x_ref, wih_ref, bih_ref, whh_ref, bhh_ref,
                  wm_ref, bm_ref, xn_ref, t_ref):
    m = m2_ref[0] + m2_ref[1]
    xn = _gru(m, x_ref[...], wih_ref, bih_ref, whh_ref, bhh_ref)
    xn_ref[...] = xn
    t_ref[...] = (jnp.dot(xn, wm_ref[...],
                          preferred_element_type=jnp.float32) + bm_ref[...])


_gru_msg_tc = pl.pallas_call(
    _gru_msg_body,
    grid=(NB,),
    in_specs=[
        pl.BlockSpec((NC, BN, H), lambda i: (0, i, 0)),
        pl.BlockSpec((BN, H), lambda i: (i, 0)),
        pl.BlockSpec((H, 3 * H), lambda i: (0, 0)),
        pl.BlockSpec((1, 3 * H), lambda i: (0, 0)),
        pl.BlockSpec((H, 3 * H), lambda i: (0, 0)),
        pl.BlockSpec((1, 3 * H), lambda i: (0, 0)),
        pl.BlockSpec((H, H), lambda i: (0, 0)),
        pl.BlockSpec((1, H), lambda i: (0, 0)),
    ],
    out_specs=[
        pl.BlockSpec((BN, H), lambda i: (i, 0)),
        pl.BlockSpec((BN, H), lambda i: (i, 0)),
    ],
    out_shape=[
        jax.ShapeDtypeStruct((N, H), jnp.float32),
        jax.ShapeDtypeStruct((N, H), jnp.float32),
    ],
)


def _gru_read_body(m2_ref, x_ref, wih_ref, bih_ref, whh_ref, bhh_ref,
                   wc_ref, bc_ref, out_ref, maxv):
    i = pl.program_id(0)
    m = m2_ref[0] + m2_ref[1]
    xn = _gru(m, x_ref[...], wih_ref, bih_ref, whh_ref, bhh_ref)
    part = jnp.max(xn, axis=0, keepdims=True)

    @pl.when(i == 0)
    def _init():
        maxv[...] = part

    @pl.when(i > 0)
    def _acc():
        maxv[...] = jnp.maximum(maxv[...], part)

    @pl.when(i == NB - 1)
    def _fin():
        out_ref[...] = (jnp.dot(maxv[...], wc_ref[...],
                                preferred_element_type=jnp.float32) + bc_ref[...])


_gru_read_tc = pl.pallas_call(
    _gru_read_body,
    grid=(NB,),
    in_specs=[
        pl.BlockSpec((NC, BN, H), lambda i: (0, i, 0)),
        pl.BlockSpec((BN, H), lambda i: (i, 0)),
        pl.BlockSpec((H, 3 * H), lambda i: (0, 0)),
        pl.BlockSpec((1, 3 * H), lambda i: (0, 0)),
        pl.BlockSpec((H, 3 * H), lambda i: (0, 0)),
        pl.BlockSpec((1, 3 * H), lambda i: (0, 0)),
        pl.BlockSpec((H, H), lambda i: (0, 0)),
        pl.BlockSpec((1, H), lambda i: (0, 0)),
    ],
    out_specs=pl.BlockSpec((1, H), lambda i: (0, 0)),
    out_shape=jax.ShapeDtypeStruct((1, H), jnp.float32),
    scratch_shapes=[pltpu.VMEM((1, H), jnp.float32)],
)


def kernel(h, edge_index, W_msg, b_msg, W_ih, b_ih, W_hh, b_hh, W_cls, b_cls):
    src = edge_index[0]
    dst = edge_index[1]
    pad = E_PAD - E
    srcp = jnp.concatenate([src, jnp.zeros((pad,), jnp.int32)]).reshape(NCHUNKS, CHUNK)
    dst_pad = N + (jnp.arange(pad, dtype=jnp.int32) % (N_OUT - N))
    dstp = jnp.concatenate([dst, dst_pad]).reshape(NCHUNKS, CHUNK)

    wm = W_msg.T
    wih = W_ih.T
    whh = W_hh.T
    bm = b_msg.reshape(1, H)
    bih = b_ih.reshape(1, 3 * H)
    bhh = b_hh.reshape(1, 3 * H)
    wc = jnp.pad(W_cls.T, ((0, 0), (0, H - C)))
    bc = jnp.pad(b_cls.reshape(1, C), ((0, 0), (0, H - C)))

    x = h
    t = _msg_tc(x, wm, bm)
    for step in range(STEPS):
        m2 = _sc_scatter_kernel()(t, srcp, dstp)
        if step < STEPS - 1:
            x, t = _gru_msg_tc(m2, x, wih, bih, whh, bhh, wm, bm)
        else:
            logits = _gru_read_tc(m2, x, wih, bih, whh, bhh, wc, bc)
    return logits[:, :C]
